# baseline (speedup 1.0000x reference)
import jax
import jax.numpy as jnp
from jax.experimental import pallas as pl
from jax.experimental.pallas import tpu as pltpu

_SIZES = (504, 2496, 2000, 2000, 1496, 1000, 504)
_OFFS = tuple(sum(_SIZES[:i]) for i in range(len(_SIZES)))
_N = len(_SIZES)


def _linear_kernel(x_hbm, w_ref, b_ref, o_hbm, *rest):
    bufs = rest[0:_N]
    lsem = rest[_N]
    ssem = rest[_N + 1]

    def load(i):
        return pltpu.make_async_copy(
            x_hbm.at[pl.ds(_OFFS[i], _SIZES[i]), :], bufs[i], lsem.at[i])

    def store(i):
        return pltpu.make_async_copy(
            bufs[i], o_hbm.at[pl.ds(_OFFS[i], _SIZES[i]), :], ssem.at[i])

    for i in range(_N):
        load(i).start()

    w_scaled = 0.1 * w_ref[...]
    b_scaled = 0.1 * b_ref[...]

    for i in range(_N):
        load(i).wait()
        x_blk = bufs[i][...]
        y = jax.lax.dot_general(
            x_blk, w_scaled,
            dimension_numbers=(((1,), (1,)), ((), ())),
            preferred_element_type=jnp.float32,
        )
        bufs[i][...] = x_blk + (y + b_scaled)
        store(i).start()

    for i in range(_N):
        store(i).wait()


def kernel(x, W, b):
    m, a = x.shape
    b2d = b.reshape(1, a)
    return pl.pallas_call(
        _linear_kernel,
        in_specs=[
            pl.BlockSpec(memory_space=pl.ANY),
            pl.BlockSpec(memory_space=pltpu.MemorySpace.VMEM),
            pl.BlockSpec(memory_space=pltpu.MemorySpace.VMEM),
        ],
        out_specs=pl.BlockSpec(memory_space=pl.ANY),
        out_shape=jax.ShapeDtypeStruct((m, a), x.dtype),
        scratch_shapes=(
            [pltpu.VMEM((s, a), jnp.float32) for s in _SIZES]
            + [pltpu.SemaphoreType.DMA((_N,)),
               pltpu.SemaphoreType.DMA((_N,))]
        ),
    )(x, W, b2d)


# speedup vs baseline: 1.0293x; 1.0293x over previous
import jax
import jax.numpy as jnp
from jax.experimental import pallas as pl
from jax.experimental.pallas import tpu as pltpu

_SIZES = (2000, 2000, 2000, 2000, 1000, 800, 200)
_OFFS = tuple(sum(_SIZES[:i]) for i in range(len(_SIZES)))
_N = len(_SIZES)


def _linear_kernel(x_hbm, w_ref, b_ref, o_hbm, *rest):
    bufs = rest[0:_N]
    lsem = rest[_N]
    ssem = rest[_N + 1]

    def load(i):
        return pltpu.make_async_copy(
            x_hbm.at[pl.ds(_OFFS[i], _SIZES[i]), :], bufs[i], lsem.at[i])

    def store(i):
        return pltpu.make_async_copy(
            bufs[i], o_hbm.at[pl.ds(_OFFS[i], _SIZES[i]), :], ssem.at[i])

    for i in range(_N):
        load(i).start()

    w_scaled = 0.1 * w_ref[...]
    b_scaled = 0.1 * b_ref[...]

    for i in range(_N):
        load(i).wait()
        x_blk = bufs[i][...]
        y = jax.lax.dot_general(
            x_blk, w_scaled,
            dimension_numbers=(((1,), (1,)), ((), ())),
            preferred_element_type=jnp.float32,
        )
        bufs[i][...] = x_blk + (y + b_scaled)
        store(i).start()

    for i in range(_N):
        store(i).wait()


def kernel(x, W, b):
    m, a = x.shape
    b2d = b.reshape(1, a)
    return pl.pallas_call(
        _linear_kernel,
        in_specs=[
            pl.BlockSpec(memory_space=pl.ANY),
            pl.BlockSpec(memory_space=pltpu.MemorySpace.VMEM),
            pl.BlockSpec(memory_space=pltpu.MemorySpace.VMEM),
        ],
        out_specs=pl.BlockSpec(memory_space=pl.ANY),
        out_shape=jax.ShapeDtypeStruct((m, a), x.dtype),
        scratch_shapes=(
            [pltpu.VMEM((s, a), jnp.float32) for s in _SIZES]
            + [pltpu.SemaphoreType.DMA((_N,)),
               pltpu.SemaphoreType.DMA((_N,))]
        ),
    )(x, W, b2d)
